# baseline (speedup 1.0000x reference)
import functools

import numpy as np
import jax
import jax.numpy as jnp
from jax import lax
from jax.experimental import pallas as pl
from jax.experimental.pallas import tpu as pltpu
from jax.experimental.pallas import tpu_sc as plsc

_B, _N, _K, _C, _DEPTH, _G, _H = 2, 10000, 16, 128, 2, 32, 256
_M = _B * _N
_SC_NC = 2
_NW = _SC_NC * 16
_RPW = 20480 // _NW
_NPAD = _NW * _RPW
_S = 8
_R = 2000
_NB = _M // _R

_PERM = np.array(
    [((q % 2) * 16 + (q % _G) // 2) * 4 + q // _G for q in range(_C)],
    np.int32)
_PMAT = np.zeros((_C, _C), np.float32)
_PMAT[np.arange(_C), _PERM] = 1.0

_F32 = jnp.float32
_BF16 = jnp.bfloat16
_TW = _C + 16


def _accum_stats(st_ref, v, i):
    @pl.when(i == 0)
    def _():
        st_ref[...] = jnp.zeros_like(st_ref)

    s0 = jnp.sum(v, axis=0, keepdims=True)
    s1 = jnp.sum(v * v, axis=0, keepdims=True)
    st_ref[...] += jnp.concatenate(
        [s0, s1, jnp.zeros((6, _C), _F32)], axis=0)


def _bn_from_stats(st_ref, v_blk, g_ref, b_ref):
    mu = st_ref[0:1, :] * (1.0 / _M)
    var = st_ref[1:2, :] * (1.0 / _M) - mu * mu
    inv = lax.rsqrt(var + 1e-5)
    return (v_blk - mu) * (inv * g_ref[...]) + b_ref[...]


def _mlp_body(has_proj, *args):
    if has_proj:
        (x_ref, w1t_ref, b1_ref, w2t_ref, g_ref, b_ref, pw_ref, xyzb_ref,
         o_ref, p_ref, h2_scr, st_scr) = args
    else:
        (x_ref, w1t_ref, b1_ref, w2t_ref, g_ref, b_ref,
         o_ref, h2_scr, st_scr) = args
    ph = pl.program_id(0)
    i = pl.program_id(1)

    @pl.when(ph == 0)
    def _():
        h = jax.nn.gelu(
            jnp.dot(x_ref[...], w1t_ref[...], preferred_element_type=_F32)
            + b1_ref[...])
        h2 = jnp.dot(h, w2t_ref[...], preferred_element_type=_F32)
        h2_scr[pl.ds(i * _R, _R), :] = h2
        _accum_stats(st_scr, h2, i)

    @pl.when(ph == 1)
    def _():
        h2 = h2_scr[pl.ds(i * _R, _R), :]
        y = x_ref[...] + _bn_from_stats(st_scr, h2, g_ref, b_ref)
        o_ref[...] = y
        if has_proj:
            p_ref[:, :_C] = jnp.dot(
                y, pw_ref[...], preferred_element_type=_F32).astype(_BF16)
            p_ref[:, _C:] = xyzb_ref[...]


def _mlp_block(x2d, w1t, b1, w2t, g, b, pwT=None, xyzb=None):
    has_proj = pwT is not None
    in_specs = [
        pl.BlockSpec((_R, _C), lambda p, i: (i, 0)),
        pl.BlockSpec((_C, _H), lambda p, i: (0, 0)),
        pl.BlockSpec((1, _H), lambda p, i: (0, 0)),
        pl.BlockSpec((_H, _C), lambda p, i: (0, 0)),
        pl.BlockSpec((1, _C), lambda p, i: (0, 0)),
        pl.BlockSpec((1, _C), lambda p, i: (0, 0)),
    ]
    out_spec = pl.BlockSpec((_R, _C), lambda p, i: (i * p, 0))
    scratch = [pltpu.VMEM((_M, _C), _F32), pltpu.VMEM((8, _C), _F32)]
    if has_proj:
        return pl.pallas_call(
            functools.partial(_mlp_body, True),
            grid=(2, _NB),
            in_specs=in_specs + [
                pl.BlockSpec((_C, _C), lambda p, i: (0, 0)),
                pl.BlockSpec((_R, 16), lambda p, i: (i, 0))],
            out_specs=[out_spec,
                       pl.BlockSpec((_R, _TW), lambda p, i: (i * p, 0))],
            out_shape=[jax.ShapeDtypeStruct((_M, _C), _F32),
                       jax.ShapeDtypeStruct((_M, _TW), _BF16)],
            scratch_shapes=scratch,
        )(x2d, w1t, b1, w2t, g, b, pwT, xyzb)
    return pl.pallas_call(
        functools.partial(_mlp_body, False),
        grid=(2, _NB),
        in_specs=in_specs,
        out_specs=out_spec,
        out_shape=jax.ShapeDtypeStruct((_M, _C), _F32),
        scratch_shapes=scratch,
    )(x2d, w1t, b1, w2t, g, b)


def _lfp_body(has_proj, *args):
    if has_proj:
        (x_ref, a_ref, g_ref, b_ref, pm_ref, pw_ref, xyzb_ref,
         o_ref, p_ref, st_scr) = args
    else:
        (x_ref, a_ref, g_ref, b_ref, pm_ref, o_ref, st_scr) = args
    ph = pl.program_id(0)
    i = pl.program_id(1)

    @pl.when(ph == 0)
    def _():
        _accum_stats(st_scr, a_ref[...].astype(_F32), i)

    @pl.when(ph == 1)
    def _():
        z = _bn_from_stats(st_scr, a_ref[...].astype(_F32), g_ref, b_ref)
        y = x_ref[...] + jnp.dot(z, pm_ref[...], preferred_element_type=_F32)
        o_ref[...] = y
        if has_proj:
            p_ref[:, :_C] = jnp.dot(
                y, pw_ref[...], preferred_element_type=_F32).astype(_BF16)
            p_ref[:, _C:] = xyzb_ref[...]


def _lfp_block(x2d, aggp, g, b, pm, pwT=None, xyzb=None):
    has_proj = pwT is not None
    in_specs = [
        pl.BlockSpec((_R, _C), lambda p, i: (i * p, 0)),
        pl.BlockSpec((_R, _C), lambda p, i: (i, 0)),
        pl.BlockSpec((1, _C), lambda p, i: (0, 0)),
        pl.BlockSpec((1, _C), lambda p, i: (0, 0)),
        pl.BlockSpec((_C, _C), lambda p, i: (0, 0)),
    ]
    out_spec = pl.BlockSpec((_R, _C), lambda p, i: (i * p, 0))
    scratch = [pltpu.VMEM((8, _C), _F32)]
    if has_proj:
        return pl.pallas_call(
            functools.partial(_lfp_body, True),
            grid=(2, _NB),
            in_specs=in_specs + [
                pl.BlockSpec((_C, _C), lambda p, i: (0, 0)),
                pl.BlockSpec((_R, 16), lambda p, i: (i, 0))],
            out_specs=[out_spec,
                       pl.BlockSpec((_R, _TW), lambda p, i: (i * p, 0))],
            out_shape=[jax.ShapeDtypeStruct((_M, _C), _F32),
                       jax.ShapeDtypeStruct((_M, _TW), _BF16)],
            scratch_shapes=scratch,
        )(x2d, aggp, g, b, pm, pwT, xyzb)
    return pl.pallas_call(
        functools.partial(_lfp_body, False),
        grid=(2, _NB),
        in_specs=in_specs,
        out_specs=out_spec,
        out_shape=jax.ShapeDtypeStruct((_M, _C), _F32),
        scratch_shapes=scratch,
    )(x2d, aggp, g, b, pm)


_EPC = _S * _K
_NCH = _RPW // _S


def _sc_agg_body(table_h, xyzp_h, knnf_h, c_h, o_h,
                 knn_all, dst_all,
                 rows_a, rows_b, rows_c, rows_d,
                 out_a, out_b, out_c, out_d, cv,
                 gsem_a, gsem_b, gsem_c, gsem_d,
                 osem_a, osem_b, osem_c, osem_d):
    cid = lax.axis_index("c")
    sid = lax.axis_index("s")
    wid = sid * _SC_NC + cid
    w0 = wid * _RPW
    pltpu.sync_copy(c_h, cv)
    pltpu.sync_copy(knnf_h.at[pl.ds(w0 * _K, _RPW * _K)], knn_all)
    pltpu.sync_copy(xyzp_h.at[pl.ds(w0, _RPW)], dst_all)
    a_lo, a_hi = cv[0, :], cv[1, :]
    bx_lo, bx_hi = cv[2, :], cv[3, :]
    by_lo, by_hi = cv[4, :], cv[5, :]
    bz_lo, bz_hi = cv[6, :], cv[7, :]
    ns_lo, ns_hi = cv[8, :], cv[9, :]

    def start_gathers(c, rows_v, sem):
        idx = knn_all.at[pl.ds(c * _EPC, _EPC)]
        pltpu.async_copy(table_h.at[idx], rows_v, sem)

    def wait_gathers(rows_v, sem):
        idx = knn_all.at[pl.ds(0, _EPC)]
        pltpu.make_async_copy(table_h.at[idx], rows_v, sem).wait()

    def wait_out(out_v, sem):
        pltpu.make_async_copy(out_v, o_h.at[pl.ds(w0, _S)], sem).wait()

    def compute(c, rows_v, out_v):
        base_l = c * _S

        @pl.loop(0, _S)
        def _node(n):
            d_row = dst_all[base_l + n, :]
            accs = [None] * 4
            for k in range(_K):
                e = n * _K + k
                cw = rows_v[e, pl.ds(_C - 16, 32)]
                ca, cb = plsc.unpack(cw, format=plsc.PackFormat.INTERLEAVED)
                px = ca[8] - d_row[0]
                py = cb[8] - d_row[1]
                pz = ca[9] - d_row[2]
                r2 = px * px + py * py + pz * pz
                t_lo = a_lo + bx_lo * px + by_lo * py + bz_lo * pz + ns_lo * r2
                t_hi = a_hi + bx_hi * px + by_hi * py + bz_hi * pz + ns_hi * r2
                w_lo = jnp.exp(t_lo)
                w_hi = jnp.exp(t_hi)
                wpk = plsc.pack(w_lo, w_hi,
                                format=plsc.PackFormat.INTERLEAVED)
                for j in range(4):
                    row = rows_v[e, pl.ds(j * 32, 32)]
                    v = row * wpk
                    accs[j] = v if k == 0 else jnp.maximum(accs[j], v)
            for j in range(4):
                out_v[n, pl.ds(j * 32, 32)] = accs[j]

    rows_bufs = [rows_a, rows_b, rows_c, rows_d]
    out_bufs = [out_a, out_b, out_c, out_d]
    gsems = [gsem_a, gsem_b, gsem_c, gsem_d]
    osems = [osem_a, osem_b, osem_c, osem_d]
    nbuf = 4

    for b in range(nbuf):
        start_gathers(b, rows_bufs[b], gsems[b])

    @pl.loop(0, _NCH // nbuf)
    def _ring(g):
        for b in range(nbuf):
            c = g * nbuf + b
            wait_gathers(rows_bufs[b], gsems[b])

            @pl.when(g > 0)
            def _():
                wait_out(out_bufs[b], osems[b])

            compute(c, rows_bufs[b], out_bufs[b])
            pltpu.async_copy(out_bufs[b], o_h.at[pl.ds(w0 + c * _S, _S)],
                             osems[b])
            start_gathers(lax.rem(c + nbuf, _NCH), rows_bufs[b], gsems[b])

    for b in range(nbuf):
        wait_gathers(rows_bufs[b], gsems[b])
        wait_out(out_bufs[b], osems[b])


def _sc_agg(table, xyzp, knnf, consts):
    mesh = plsc.VectorSubcoreMesh(core_axis_name="c", subcore_axis_name="s",
                                  num_cores=_SC_NC, num_subcores=16)
    k = pl.kernel(
        _sc_agg_body,
        compiler_params=pltpu.CompilerParams(use_tc_tiling_on_sc=False,
                                             needs_layout_passes=False),
        out_type=jax.ShapeDtypeStruct((_NPAD, _C), jnp.bfloat16),
        mesh=mesh,
        scratch_types=(
            [pltpu.VMEM((_RPW * _K,), jnp.int32),
             pltpu.VMEM((_RPW, 16), _F32)]
            + [pltpu.VMEM((_EPC, _TW), _BF16)] * 4
            + [pltpu.VMEM((_S, _C), _BF16)] * 4
            + [pltpu.VMEM((16, 16), _F32)]
            + [pltpu.SemaphoreType.DMA] * 8
        ),
    )
    return k(table, xyzp, knnf, consts)


def _lfp_consts(coor, scale):
    s2 = scale * scale
    c3 = coor.reshape(_G, 3)
    a = -s2 * jnp.sum(c3 * c3, axis=1)
    bx = 2.0 * s2 * c3[:, 0]
    by = 2.0 * s2 * c3[:, 1]
    bz = 2.0 * s2 * c3[:, 2]
    ns = -s2
    rows = jnp.concatenate([
        a.reshape(2, 16), bx.reshape(2, 16), by.reshape(2, 16),
        bz.reshape(2, 16), ns.reshape(2, 16), jnp.zeros((6, 16), _F32),
    ], axis=0)
    return rows.astype(_F32)


def kernel(x, xyz, params, knn):
    p = params
    perm = jnp.asarray(_PERM)
    pm = jnp.asarray(_PMAT)

    x2d = x.reshape(_M, _C)
    xyz2 = xyz.reshape(_M, 3).astype(_F32)
    xyzp = jnp.zeros((_NPAD, 16), _F32).at[:_M, :3].set(xyz2)
    offs = (jnp.arange(_B, dtype=jnp.int32) * _N).reshape(_B, 1, 1)
    knnf2 = (knn.astype(jnp.int32) + offs).reshape(_M, _K)
    knnf = jnp.zeros((_NPAD, _K), jnp.int32).at[:_M].set(knnf2).reshape(-1)
    xyzb = xyzp[:_M].astype(_BF16)

    w1t0, b10, w2t0 = (p['mlp0_w1'].T, p['mlp0_b1'].reshape(1, _H),
                       p['mlp0_w2'].T)
    w1tm, b1m, w2tm = (p['mlpm_w1'].T, p['mlpm_b1'].reshape(1, _H),
                       p['mlpm_w2'].T)
    g0, b0 = p['mlp0_bnw'].reshape(1, _C), p['mlp0_bnb'].reshape(1, _C)
    gm, bm = p['mlpm_bnw'].reshape(1, _C), p['mlpm_bnb'].reshape(1, _C)

    pwT = [p['lfp_proj_w'][i][perm, :].T for i in range(_DEPTH)]
    glp = [p['lfp_bn_w'][i][perm].reshape(1, _C) for i in range(_DEPTH)]
    blp = [p['lfp_bn_b'][i][perm].reshape(1, _C) for i in range(_DEPTH)]
    consts = [_lfp_consts(p['lfp_coor'][i], p['lfp_scale'][i])
              for i in range(_DEPTH)]

    x1, xp0 = _mlp_block(x2d, w1t0, b10, w2t0, g0, b0, pwT[0], xyzb)

    agg0 = _sc_agg(xp0, xyzp, knnf, consts[0])
    x2, xp1 = _lfp_block(x1, agg0, glp[0], blp[0], pm, pwT[1], xyzb)

    agg1 = _sc_agg(xp1, xyzp, knnf, consts[1])
    x3 = _lfp_block(x2, agg1, glp[1], blp[1], pm)

    x4 = _mlp_block(x3, w1tm, b1m, w2tm, gm, bm)

    return x4.reshape(_B, _N, _C)


# speedup vs baseline: 1.2291x; 1.2291x over previous
import functools

import numpy as np
import jax
import jax.numpy as jnp
from jax import lax
from jax.experimental import pallas as pl
from jax.experimental.pallas import tpu as pltpu
from jax.experimental.pallas import tpu_sc as plsc

_B, _N, _K, _C, _DEPTH, _G, _H = 2, 10000, 16, 128, 2, 32, 256
_M = _B * _N
_SC_NC = 2
_NW = _SC_NC * 16
_RPW = 20480 // _NW
_NPAD = _NW * _RPW
_S = 8
_R = 2000
_NB = _M // _R

_PERM = np.array(
    [((q % 2) * 16 + (q % _G) // 2) * 4 + q // _G for q in range(_C)],
    np.int32)
_PMAT = np.zeros((_C, _C), np.float32)
_PMAT[np.arange(_C), _PERM] = 1.0

_F32 = jnp.float32
_BF16 = jnp.bfloat16
_TW = _C + 16


def _accum_stats(st_ref, v, i):
    @pl.when(i == 0)
    def _():
        st_ref[...] = jnp.zeros_like(st_ref)

    s0 = jnp.sum(v, axis=0, keepdims=True)
    s1 = jnp.sum(v * v, axis=0, keepdims=True)
    st_ref[...] += jnp.concatenate(
        [s0, s1, jnp.zeros((6, _C), _F32)], axis=0)


def _bn_from_stats(st_ref, v_blk, g_ref, b_ref):
    mu = st_ref[0:1, :] * (1.0 / _M)
    var = st_ref[1:2, :] * (1.0 / _M) - mu * mu
    inv = lax.rsqrt(var + 1e-5)
    return (v_blk - mu) * (inv * g_ref[...]) + b_ref[...]


def _mlp_body(has_proj, *args):
    if has_proj:
        (x_ref, w1t_ref, b1_ref, w2t_ref, g_ref, b_ref, pw_ref,
         o_ref, p_ref, h2_scr, st_scr) = args
    else:
        (x_ref, w1t_ref, b1_ref, w2t_ref, g_ref, b_ref,
         o_ref, h2_scr, st_scr) = args
    ph = pl.program_id(0)
    i = pl.program_id(1)

    @pl.when(ph == 0)
    def _():
        h = jax.nn.gelu(
            jnp.dot(x_ref[...], w1t_ref[...], preferred_element_type=_F32)
            + b1_ref[...])
        h2 = jnp.dot(h, w2t_ref[...], preferred_element_type=_F32)
        h2_scr[pl.ds(i * _R, _R), :] = h2
        _accum_stats(st_scr, h2, i)

    @pl.when(ph == 1)
    def _():
        h2 = h2_scr[pl.ds(i * _R, _R), :]
        y = x_ref[...] + _bn_from_stats(st_scr, h2, g_ref, b_ref)
        o_ref[...] = y
        if has_proj:
            p_ref[...] = jnp.dot(
                y, pw_ref[...], preferred_element_type=_F32).astype(_BF16)


def _mlp_block(x2d, w1t, b1, w2t, g, b, pwT=None):
    has_proj = pwT is not None
    in_specs = [
        pl.BlockSpec((_R, _C), lambda p, i: (i, 0)),
        pl.BlockSpec((_C, _H), lambda p, i: (0, 0)),
        pl.BlockSpec((1, _H), lambda p, i: (0, 0)),
        pl.BlockSpec((_H, _C), lambda p, i: (0, 0)),
        pl.BlockSpec((1, _C), lambda p, i: (0, 0)),
        pl.BlockSpec((1, _C), lambda p, i: (0, 0)),
    ]
    out_spec = pl.BlockSpec((_R, _C), lambda p, i: (i * p, 0))
    scratch = [pltpu.VMEM((_M, _C), _F32), pltpu.VMEM((8, _C), _F32)]
    if has_proj:
        return pl.pallas_call(
            functools.partial(_mlp_body, True),
            grid=(2, _NB),
            in_specs=in_specs + [pl.BlockSpec((_C, _C), lambda p, i: (0, 0))],
            out_specs=[out_spec, out_spec],
            out_shape=[jax.ShapeDtypeStruct((_M, _C), _F32),
                       jax.ShapeDtypeStruct((_M, _C), _BF16)],
            scratch_shapes=scratch,
        )(x2d, w1t, b1, w2t, g, b, pwT)
    return pl.pallas_call(
        functools.partial(_mlp_body, False),
        grid=(2, _NB),
        in_specs=in_specs,
        out_specs=out_spec,
        out_shape=jax.ShapeDtypeStruct((_M, _C), _F32),
        scratch_shapes=scratch,
    )(x2d, w1t, b1, w2t, g, b)


def _lfp_body(has_proj, *args):
    if has_proj:
        (x_ref, a_ref, g_ref, b_ref, pm_ref, pw_ref,
         o_ref, p_ref, st_scr) = args
    else:
        (x_ref, a_ref, g_ref, b_ref, pm_ref, o_ref, st_scr) = args
    ph = pl.program_id(0)
    i = pl.program_id(1)

    @pl.when(ph == 0)
    def _():
        _accum_stats(st_scr, a_ref[...].astype(_F32), i)

    @pl.when(ph == 1)
    def _():
        z = _bn_from_stats(st_scr, a_ref[...].astype(_F32), g_ref, b_ref)
        y = x_ref[...] + jnp.dot(z, pm_ref[...], preferred_element_type=_F32)
        o_ref[...] = y
        if has_proj:
            p_ref[...] = jnp.dot(
                y, pw_ref[...], preferred_element_type=_F32).astype(_BF16)


def _lfp_block(x2d, aggp, g, b, pm, pwT=None):
    has_proj = pwT is not None
    in_specs = [
        pl.BlockSpec((_R, _C), lambda p, i: (i * p, 0)),
        pl.BlockSpec((_R, _C), lambda p, i: (i, 0)),
        pl.BlockSpec((1, _C), lambda p, i: (0, 0)),
        pl.BlockSpec((1, _C), lambda p, i: (0, 0)),
        pl.BlockSpec((_C, _C), lambda p, i: (0, 0)),
    ]
    out_spec = pl.BlockSpec((_R, _C), lambda p, i: (i * p, 0))
    scratch = [pltpu.VMEM((8, _C), _F32)]
    if has_proj:
        return pl.pallas_call(
            functools.partial(_lfp_body, True),
            grid=(2, _NB),
            in_specs=in_specs + [pl.BlockSpec((_C, _C), lambda p, i: (0, 0))],
            out_specs=[out_spec, out_spec],
            out_shape=[jax.ShapeDtypeStruct((_M, _C), _F32),
                       jax.ShapeDtypeStruct((_M, _C), _BF16)],
            scratch_shapes=scratch,
        )(x2d, aggp, g, b, pm, pwT)
    return pl.pallas_call(
        functools.partial(_lfp_body, False),
        grid=(2, _NB),
        in_specs=in_specs,
        out_specs=out_spec,
        out_shape=jax.ShapeDtypeStruct((_M, _C), _F32),
        scratch_shapes=scratch,
    )(x2d, aggp, g, b, pm)


_EPC = _S * _K
_NCH = _RPW // _S


def _sc_agg_body(table_h, xyzp_h, knnf_h, c_h, o_h,
                 knn_all, dst_all,
                 rows_a, rows_b, rows_c, rows_d,
                 pn_a, pn_b, pn_c, pn_d,
                 out_a, out_b, out_c, out_d, cv,
                 gsem_a, gsem_b, gsem_c, gsem_d,
                 osem_a, osem_b, osem_c, osem_d):
    cid = lax.axis_index("c")
    sid = lax.axis_index("s")
    wid = sid * _SC_NC + cid
    w0 = wid * _RPW
    pltpu.sync_copy(c_h, cv)
    pltpu.sync_copy(knnf_h.at[pl.ds(w0 * _K, _RPW * _K)], knn_all)
    pltpu.sync_copy(xyzp_h.at[pl.ds(w0, _RPW)], dst_all)
    a_lo, a_hi = cv[0, :], cv[1, :]
    bx_lo, bx_hi = cv[2, :], cv[3, :]
    by_lo, by_hi = cv[4, :], cv[5, :]
    bz_lo, bz_hi = cv[6, :], cv[7, :]
    ns_lo, ns_hi = cv[8, :], cv[9, :]

    def start_gathers(c, rows_v, pn_v, sem):
        idx = knn_all.at[pl.ds(c * _EPC, _EPC)]
        pltpu.async_copy(table_h.at[idx], rows_v, sem)
        pltpu.async_copy(xyzp_h.at[idx], pn_v, sem)

    def wait_gathers(rows_v, pn_v, sem):
        idx = knn_all.at[pl.ds(0, _EPC)]
        pltpu.make_async_copy(table_h.at[idx], rows_v, sem).wait()
        pltpu.make_async_copy(xyzp_h.at[idx], pn_v, sem).wait()

    def wait_out(out_v, sem):
        pltpu.make_async_copy(out_v, o_h.at[pl.ds(w0, _S)], sem).wait()

    def compute(c, rows_v, pn_v, out_v):
        base_l = c * _S

        @pl.loop(0, _S)
        def _node(n):
            d_row = dst_all[base_l + n, :]
            accs = [None] * 4
            for k in range(_K):
                e = n * _K + k
                rel = pn_v[e, :] - d_row
                px = rel[0]
                py = rel[1]
                pz = rel[2]
                r2 = px * px + py * py + pz * pz
                t_lo = a_lo + bx_lo * px + by_lo * py + bz_lo * pz + ns_lo * r2
                t_hi = a_hi + bx_hi * px + by_hi * py + bz_hi * pz + ns_hi * r2
                w_lo = jnp.exp(t_lo)
                w_hi = jnp.exp(t_hi)
                wpk = plsc.pack(w_lo, w_hi,
                                format=plsc.PackFormat.INTERLEAVED)
                for j in range(4):
                    row = rows_v[e, pl.ds(j * 32, 32)]
                    v = row * wpk
                    accs[j] = v if k == 0 else jnp.maximum(accs[j], v)
            for j in range(4):
                out_v[n, pl.ds(j * 32, 32)] = accs[j]

    rows_bufs = [rows_a, rows_b, rows_c, rows_d]
    pn_bufs = [pn_a, pn_b, pn_c, pn_d]
    out_bufs = [out_a, out_b, out_c, out_d]
    gsems = [gsem_a, gsem_b, gsem_c, gsem_d]
    osems = [osem_a, osem_b, osem_c, osem_d]
    nbuf = 4

    for b in range(nbuf):
        start_gathers(b, rows_bufs[b], pn_bufs[b], gsems[b])

    @pl.loop(0, _NCH // nbuf)
    def _ring(g):
        for b in range(nbuf):
            c = g * nbuf + b
            wait_gathers(rows_bufs[b], pn_bufs[b], gsems[b])

            @pl.when(g > 0)
            def _():
                wait_out(out_bufs[b], osems[b])

            compute(c, rows_bufs[b], pn_bufs[b], out_bufs[b])
            pltpu.async_copy(out_bufs[b], o_h.at[pl.ds(w0 + c * _S, _S)],
                             osems[b])
            start_gathers(lax.rem(c + nbuf, _NCH),
                          rows_bufs[b], pn_bufs[b], gsems[b])

    for b in range(nbuf):
        wait_gathers(rows_bufs[b], pn_bufs[b], gsems[b])
        wait_out(out_bufs[b], osems[b])


def _sc_agg(table, xyzp, knnf, consts):
    mesh = plsc.VectorSubcoreMesh(core_axis_name="c", subcore_axis_name="s",
                                  num_cores=_SC_NC, num_subcores=16)
    k = pl.kernel(
        _sc_agg_body,
        compiler_params=pltpu.CompilerParams(use_tc_tiling_on_sc=False,
                                             needs_layout_passes=False),
        out_type=jax.ShapeDtypeStruct((_NPAD, _C), jnp.bfloat16),
        mesh=mesh,
        scratch_types=(
            [pltpu.VMEM((_RPW * _K,), jnp.int32),
             pltpu.VMEM((_RPW, 16), _F32)]
            + [pltpu.VMEM((_EPC, _C), _BF16)] * 4
            + [pltpu.VMEM((_EPC, 16), _F32)] * 4
            + [pltpu.VMEM((_S, _C), _BF16)] * 4
            + [pltpu.VMEM((16, 16), _F32)]
            + [pltpu.SemaphoreType.DMA] * 8
        ),
    )
    return k(table, xyzp, knnf, consts)


def _lfp_consts(coor, scale):
    s2 = scale * scale
    c3 = coor.reshape(_G, 3)
    a = -s2 * jnp.sum(c3 * c3, axis=1)
    bx = 2.0 * s2 * c3[:, 0]
    by = 2.0 * s2 * c3[:, 1]
    bz = 2.0 * s2 * c3[:, 2]
    ns = -s2
    rows = jnp.concatenate([
        a.reshape(2, 16), bx.reshape(2, 16), by.reshape(2, 16),
        bz.reshape(2, 16), ns.reshape(2, 16), jnp.zeros((6, 16), _F32),
    ], axis=0)
    return rows.astype(_F32)


def kernel(x, xyz, params, knn):
    p = params
    perm = jnp.asarray(_PERM)
    pm = jnp.asarray(_PMAT)

    x2d = x.reshape(_M, _C)
    xyz2 = xyz.reshape(_M, 3).astype(_F32)
    xyzp = jnp.zeros((_NPAD, 16), _F32).at[:_M, :3].set(xyz2)
    offs = (jnp.arange(_B, dtype=jnp.int32) * _N).reshape(_B, 1, 1)
    knnf2 = (knn.astype(jnp.int32) + offs).reshape(_M, _K)
    knnf = jnp.zeros((_NPAD, _K), jnp.int32).at[:_M].set(knnf2).reshape(-1)

    w1t0, b10, w2t0 = (p['mlp0_w1'].T, p['mlp0_b1'].reshape(1, _H),
                       p['mlp0_w2'].T)
    w1tm, b1m, w2tm = (p['mlpm_w1'].T, p['mlpm_b1'].reshape(1, _H),
                       p['mlpm_w2'].T)
    g0, b0 = p['mlp0_bnw'].reshape(1, _C), p['mlp0_bnb'].reshape(1, _C)
    gm, bm = p['mlpm_bnw'].reshape(1, _C), p['mlpm_bnb'].reshape(1, _C)

    pwT = [p['lfp_proj_w'][i][perm, :].T for i in range(_DEPTH)]
    glp = [p['lfp_bn_w'][i][perm].reshape(1, _C) for i in range(_DEPTH)]
    blp = [p['lfp_bn_b'][i][perm].reshape(1, _C) for i in range(_DEPTH)]
    consts = [_lfp_consts(p['lfp_coor'][i], p['lfp_scale'][i])
              for i in range(_DEPTH)]

    x1, xp0 = _mlp_block(x2d, w1t0, b10, w2t0, g0, b0, pwT[0])

    agg0 = _sc_agg(xp0, xyzp, knnf, consts[0])
    x2, xp1 = _lfp_block(x1, agg0, glp[0], blp[0], pm, pwT[1])

    agg1 = _sc_agg(xp1, xyzp, knnf, consts[1])
    x3 = _lfp_block(x2, agg1, glp[1], blp[1], pm)

    x4 = _mlp_block(x3, w1tm, b1m, w2tm, gm, bm)

    return x4.reshape(_B, _N, _C)
